# Initial kernel scaffold; baseline (speedup 1.0000x reference)
import jax
import jax.numpy as jnp
from jax.experimental import pallas as pl


def kernel(xyz_proj, points_proj, xyz_sampled_proj, W0, b0, g0, be0, W1, b1, g1, be1, W2, b2, g2, be2):
    raise NotImplementedError("write your pallas kernel here")


# speedup vs baseline: 25.3314x; 25.3314x over previous
import functools
import jax
import jax.numpy as jnp
from jax import lax
from jax.experimental import pallas as pl
from jax.experimental.pallas import tpu as pltpu
from jax.experimental.pallas import tpu_sc as plsc

K_SAMPLE = 16
KH, KW = 7, 7
DIST = 10.0
EPS = 1e-5
_PREC = lax.Precision.HIGHEST


def _select_body(xp_ref, out_ref, *, OH, OW, W_in, H_in, zero_row, n_hw):
    b = pl.program_id(0)
    ii = lax.broadcasted_iota(jnp.int32, (OH, OW), 0)
    jj = lax.broadcasted_iota(jnp.int32, (OH, OW), 1)
    cen = [xp_ref[0, c, 2:2 + OH, 2:2 + OW] for c in range(3)]
    flat0 = (2 * ii) * W_in + 2 * jj + b * n_hw
    rank = jnp.zeros((OH, OW), jnp.int32)
    slots = [jnp.full((OH, OW), zero_row, jnp.int32) for _ in range(K_SAMPLE)]
    t = 0
    for kh in range(KH):
        d = kh - KH // 2
        pr = d & 1
        mr = (d - pr) // 2
        for kw in range(KW):
            o = kw - KW // 2
            pc = o & 1
            mc = (o - pc) // 2
            plane = (pr * 2 + pc) * 3
            dist = jnp.zeros((OH, OW), jnp.float32)
            for c in range(3):
                cand = xp_ref[0, plane + c, 2 + mr:2 + mr + OH, 2 + mc:2 + mc + OW]
                diff = cand - cen[c]
                dist = dist + diff * diff
            inb = ((2 * ii + d >= 0) & (2 * ii + d < H_in)
                   & (2 * jj + o >= 0) & (2 * jj + o < W_in))
            valid = inb & (dist < DIST)
            flat_t = flat0 + (d * W_in + o)
            for s in range(min(K_SAMPLE, t + 1)):
                slots[s] = jnp.where(valid & (rank == s), flat_t, slots[s])
            rank = rank + valid.astype(jnp.int32)
            t += 1
    for s in range(K_SAMPLE):
        out_ref[0, s] = slots[s]


def _sc_gather(table, idx, rows, cols):
    info = plsc.get_sparse_core_info()
    nw = info.num_cores * info.num_subcores
    rpw = rows // nw
    ch = 384
    nch = rpw // ch
    mesh = plsc.VectorSubcoreMesh(core_axis_name="c", subcore_axis_name="s")

    @functools.partial(
        pl.kernel,
        out_type=jax.ShapeDtypeStruct((rows, cols), jnp.float32),
        mesh=mesh,
        scratch_types=[
            pltpu.VMEM((rpw,), jnp.int32),
            pltpu.VMEM((ch, cols), jnp.float32),
            pltpu.VMEM((ch, cols), jnp.float32),
            pltpu.SemaphoreType.DMA,
            pltpu.SemaphoreType.DMA,
        ],
    )
    def gk(table_hbm, idx_hbm, out_hbm, idx_v, buf0, buf1, sem0, sem1):
        wid = lax.axis_index("s") * info.num_cores + lax.axis_index("c")
        base = wid * rpw
        pltpu.sync_copy(idx_hbm.at[pl.ds(base, rpw)], idx_v)
        bufs = (buf0, buf1)
        sems = (sem0, sem1)
        cp = pltpu.async_copy(table_hbm.at[idx_v.at[pl.ds(0, ch)]], buf0, sem0)
        for c in range(nch):
            nxt = None
            if c + 1 < nch:
                nxt = pltpu.async_copy(
                    table_hbm.at[idx_v.at[pl.ds((c + 1) * ch, ch)]],
                    bufs[(c + 1) % 2], sems[(c + 1) % 2])
            cp.wait()
            pltpu.sync_copy(bufs[c % 2], out_hbm.at[pl.ds(base + c * ch, ch)])
            cp = nxt

    return gk(table, idx)


def _mlp0_body(g_ref, nx_ref, w80_ref, w0a_ref, b0_ref, y_ref, st_ref):
    step = pl.program_id(0) * K_SAMPLE + pl.program_id(1)

    @pl.when(step == 0)
    def _():
        st_ref[...] = jnp.zeros_like(st_ref)

    t = b0_ref[...] - jnp.dot(nx_ref[...], w0a_ref[...],
                              preferred_element_type=jnp.float32, precision=_PREC)
    y = jnp.dot(g_ref[...], w80_ref[...],
                preferred_element_type=jnp.float32, precision=_PREC) + t
    y_ref[...] = y
    st_ref[0:1, :] += jnp.sum(y, axis=0, keepdims=True)
    st_ref[1:2, :] += jnp.sum(y * y, axis=0, keepdims=True)


def _mlp1_body(x_ref, a_ref, c_ref, w_ref, b_ref, y_ref, st_ref):
    step = pl.program_id(0)

    @pl.when(step == 0)
    def _():
        st_ref[...] = jnp.zeros_like(st_ref)

    x = jax.nn.relu(x_ref[...] * a_ref[...] + c_ref[...])
    y = jnp.dot(x, w_ref[...], preferred_element_type=jnp.float32,
                precision=_PREC) + b_ref[...]
    y_ref[...] = y
    st_ref[0:1, :] += jnp.sum(y, axis=0, keepdims=True)
    st_ref[1:2, :] += jnp.sum(y * y, axis=0, keepdims=True)


def _mlp2_body(x_ref, a_ref, c_ref, w_ref, b_ref, ymax_ref, ymin_ref, st_ref):
    b = pl.program_id(0)
    s = pl.program_id(1)

    @pl.when((b == 0) & (s == 0))
    def _():
        st_ref[...] = jnp.zeros_like(st_ref)

    x = jax.nn.relu(x_ref[...] * a_ref[...] + c_ref[...])
    y = jnp.dot(x, w_ref[...], preferred_element_type=jnp.float32,
                precision=_PREC) + b_ref[...]

    @pl.when(s == 0)
    def _():
        ymax_ref[...] = y
        ymin_ref[...] = y

    @pl.when(s > 0)
    def _():
        ymax_ref[...] = jnp.maximum(ymax_ref[...], y)
        ymin_ref[...] = jnp.minimum(ymin_ref[...], y)

    st_ref[0:1, :] += jnp.sum(y, axis=0, keepdims=True)
    st_ref[1:2, :] += jnp.sum(y * y, axis=0, keepdims=True)


def _finish_body(ymax_ref, ymin_ref, a_ref, c_ref, out_ref):
    a = a_ref[...]
    c = c_ref[...]
    hi = jax.nn.relu(a * ymax_ref[...] + c)
    lo = jax.nn.relu(a * ymin_ref[...] + c)
    out_ref[...] = jnp.where(a > 0, hi, lo)


def _norm_coeffs(st, count, g, be):
    mean = st[0] / count
    var = st[1] / count - mean * mean
    a = g * lax.rsqrt(var + EPS)
    c = be - mean * a
    return a.reshape(1, -1), c.reshape(1, -1)


def kernel(xyz_proj, points_proj, xyz_sampled_proj, W0, b0, g0, be0,
           W1, b1, g1, be1, W2, b2, g2, be2):
    B, H, W, _ = xyz_proj.shape
    C = points_proj.shape[3]
    h, w = xyz_sampled_proj.shape[1], xyz_sampled_proj.shape[2]
    n = h * w
    n_hw = H * W
    zero_row = B * n_hw
    R = B * n * K_SAMPLE
    D1 = W1.shape[1]
    D2 = W2.shape[1]
    CP = 128

    planes = []
    for pr in (0, 1):
        for pc in (0, 1):
            q = xyz_proj[:, pr::2, pc::2, :]
            planes.append(jnp.moveaxis(q, -1, 1))
    xp = jnp.stack(planes, 1).reshape(B, 12, h, w)
    xp = jnp.pad(xp, ((0, 0), (0, 0), (2, 2), (2, 2)))

    idx = pl.pallas_call(
        functools.partial(_select_body, OH=h, OW=w, W_in=W, H_in=H,
                          zero_row=zero_row, n_hw=n_hw),
        grid=(B,),
        in_specs=[pl.BlockSpec((1, 12, h + 4, w + 4), lambda b: (b, 0, 0, 0))],
        out_specs=pl.BlockSpec((1, K_SAMPLE, h, w), lambda b: (b, 0, 0, 0)),
        out_shape=jax.ShapeDtypeStruct((B, K_SAMPLE, h, w), jnp.int32),
    )(xp)
    idx_flat = idx.reshape(R)

    table = jnp.concatenate(
        [points_proj.reshape(B * n_hw, C),
         xyz_proj.reshape(B * n_hw, 3),
         jnp.zeros((B * n_hw, CP - C - 3), jnp.float32)], axis=1)
    table = jnp.concatenate([table, jnp.zeros((8, CP), jnp.float32)], axis=0)
    G = _sc_gather(table, idx_flat, R, CP)

    W80 = jnp.concatenate(
        [W0[3:], W0[:3], jnp.zeros((CP - C - 3, W0.shape[1]), jnp.float32)], axis=0)
    nx = xyz_sampled_proj.reshape(B * n, 3)
    D0 = W0.shape[1]
    y0, st0 = pl.pallas_call(
        _mlp0_body,
        grid=(B, K_SAMPLE),
        in_specs=[
            pl.BlockSpec((n, CP), lambda b, s: (b * K_SAMPLE + s, 0)),
            pl.BlockSpec((n, 3), lambda b, s: (b, 0)),
            pl.BlockSpec((CP, D0), lambda b, s: (0, 0)),
            pl.BlockSpec((3, D0), lambda b, s: (0, 0)),
            pl.BlockSpec((1, D0), lambda b, s: (0, 0)),
        ],
        out_specs=[
            pl.BlockSpec((n, D0), lambda b, s: (b * K_SAMPLE + s, 0)),
            pl.BlockSpec((2, D0), lambda b, s: (0, 0)),
        ],
        out_shape=[
            jax.ShapeDtypeStruct((R, D0), jnp.float32),
            jax.ShapeDtypeStruct((2, D0), jnp.float32),
        ],
    )(G, nx, W80, W0[:3], b0.reshape(1, D0))
    a0, c0 = _norm_coeffs(st0, R, g0, be0)

    y1, st1 = pl.pallas_call(
        _mlp1_body,
        grid=(B * K_SAMPLE,),
        in_specs=[
            pl.BlockSpec((n, D0), lambda i: (i, 0)),
            pl.BlockSpec((1, D0), lambda i: (0, 0)),
            pl.BlockSpec((1, D0), lambda i: (0, 0)),
            pl.BlockSpec((D0, D1), lambda i: (0, 0)),
            pl.BlockSpec((1, D1), lambda i: (0, 0)),
        ],
        out_specs=[
            pl.BlockSpec((n, D1), lambda i: (i, 0)),
            pl.BlockSpec((2, D1), lambda i: (0, 0)),
        ],
        out_shape=[
            jax.ShapeDtypeStruct((R, D1), jnp.float32),
            jax.ShapeDtypeStruct((2, D1), jnp.float32),
        ],
    )(y0, a0, c0, W1, b1.reshape(1, D1))
    a1, c1 = _norm_coeffs(st1, R, g1, be1)

    ymax, ymin, st2 = pl.pallas_call(
        _mlp2_body,
        grid=(B, K_SAMPLE),
        in_specs=[
            pl.BlockSpec((n, D1), lambda b, s: (b * K_SAMPLE + s, 0)),
            pl.BlockSpec((1, D1), lambda b, s: (0, 0)),
            pl.BlockSpec((1, D1), lambda b, s: (0, 0)),
            pl.BlockSpec((D1, D2), lambda b, s: (0, 0)),
            pl.BlockSpec((1, D2), lambda b, s: (0, 0)),
        ],
        out_specs=[
            pl.BlockSpec((n, D2), lambda b, s: (b, 0)),
            pl.BlockSpec((n, D2), lambda b, s: (b, 0)),
            pl.BlockSpec((2, D2), lambda b, s: (0, 0)),
        ],
        out_shape=[
            jax.ShapeDtypeStruct((B * n, D2), jnp.float32),
            jax.ShapeDtypeStruct((B * n, D2), jnp.float32),
            jax.ShapeDtypeStruct((2, D2), jnp.float32),
        ],
    )(y1, a1, c1, W2, b2.reshape(1, D2))
    a2, c2 = _norm_coeffs(st2, R, g2, be2)

    out = pl.pallas_call(
        _finish_body,
        grid=(B,),
        in_specs=[
            pl.BlockSpec((n, D2), lambda b: (b, 0)),
            pl.BlockSpec((n, D2), lambda b: (b, 0)),
            pl.BlockSpec((1, D2), lambda b: (0, 0)),
            pl.BlockSpec((1, D2), lambda b: (0, 0)),
        ],
        out_specs=pl.BlockSpec((n, D2), lambda b: (b, 0)),
        out_shape=jax.ShapeDtypeStruct((B * n, D2), jnp.float32),
    )(ymax, ymin, a2, c2)

    pds = out.reshape(B, n, D2)
    return pds, pds.reshape(B, h, w, D2)


# speedup vs baseline: 39.4387x; 1.5569x over previous
import functools
import jax
import jax.numpy as jnp
from jax import lax
from jax.experimental import pallas as pl
from jax.experimental.pallas import tpu as pltpu
from jax.experimental.pallas import tpu_sc as plsc

K_SAMPLE = 16
KH, KW = 7, 7
DIST = 10.0
EPS = 1e-5
_PREC = lax.Precision.DEFAULT


def _select_body(xp_ref, out_ref, *, OH, OW, W_in, H_in, zero_row, n_hw):
    b = pl.program_id(0)
    ii = lax.broadcasted_iota(jnp.int32, (OH, OW), 0)
    jj = lax.broadcasted_iota(jnp.int32, (OH, OW), 1)
    cen = [xp_ref[0, c, 2:2 + OH, 2:2 + OW] for c in range(3)]
    flat0 = (2 * ii) * W_in + 2 * jj + b * n_hw
    rank = jnp.zeros((OH, OW), jnp.int32)
    slots = [jnp.full((OH, OW), zero_row, jnp.int32) for _ in range(K_SAMPLE)]
    t = 0
    for kh in range(KH):
        d = kh - KH // 2
        pr = d & 1
        mr = (d - pr) // 2
        for kw in range(KW):
            o = kw - KW // 2
            pc = o & 1
            mc = (o - pc) // 2
            plane = (pr * 2 + pc) * 3
            dist = jnp.zeros((OH, OW), jnp.float32)
            for c in range(3):
                cand = xp_ref[0, plane + c, 2 + mr:2 + mr + OH, 2 + mc:2 + mc + OW]
                diff = cand - cen[c]
                dist = dist + diff * diff
            inb = ((2 * ii + d >= 0) & (2 * ii + d < H_in)
                   & (2 * jj + o >= 0) & (2 * jj + o < W_in))
            valid = inb & (dist < DIST)
            flat_t = flat0 + (d * W_in + o)
            for s in range(min(K_SAMPLE, t + 1)):
                slots[s] = jnp.where(valid & (rank == s), flat_t, slots[s])
            rank = rank + valid.astype(jnp.int32)
            t += 1
    for s in range(K_SAMPLE):
        out_ref[0, s] = slots[s]


def _sc_gather(table, idx, rows, cols):
    info = plsc.get_sparse_core_info()
    nw = info.num_cores * info.num_subcores
    rpw = rows // nw
    ch = 384
    nch = rpw // ch
    mesh = plsc.VectorSubcoreMesh(core_axis_name="c", subcore_axis_name="s")

    @functools.partial(
        pl.kernel,
        out_type=jax.ShapeDtypeStruct((rows, cols), jnp.float32),
        mesh=mesh,
        scratch_types=[
            pltpu.VMEM((rpw,), jnp.int32),
            pltpu.VMEM((ch, cols), jnp.float32),
            pltpu.VMEM((ch, cols), jnp.float32),
            pltpu.SemaphoreType.DMA,
            pltpu.SemaphoreType.DMA,
        ],
    )
    def gk(table_hbm, idx_hbm, out_hbm, idx_v, buf0, buf1, sem0, sem1):
        wid = lax.axis_index("s") * info.num_cores + lax.axis_index("c")
        base = wid * rpw
        pltpu.sync_copy(idx_hbm.at[pl.ds(base, rpw)], idx_v)
        bufs = (buf0, buf1)
        sems = (sem0, sem1)
        cp = pltpu.async_copy(table_hbm.at[idx_v.at[pl.ds(0, ch)]], buf0, sem0)
        for c in range(nch):
            nxt = None
            if c + 1 < nch:
                nxt = pltpu.async_copy(
                    table_hbm.at[idx_v.at[pl.ds((c + 1) * ch, ch)]],
                    bufs[(c + 1) % 2], sems[(c + 1) % 2])
            cp.wait()
            pltpu.sync_copy(bufs[c % 2], out_hbm.at[pl.ds(base + c * ch, ch)])
            cp = nxt

    return gk(table, idx)


def _mlp0_body(g_ref, nx_ref, w80_ref, w0a_ref, b0_ref, y_ref, st_ref):
    step = pl.program_id(0) * K_SAMPLE + pl.program_id(1)

    @pl.when(step == 0)
    def _():
        st_ref[...] = jnp.zeros_like(st_ref)

    t = b0_ref[...] - jnp.dot(nx_ref[...], w0a_ref[...],
                              preferred_element_type=jnp.float32, precision=_PREC)
    y = jnp.dot(g_ref[...], w80_ref[...],
                preferred_element_type=jnp.float32, precision=_PREC) + t
    y_ref[...] = y
    st_ref[0:1, :] += jnp.sum(y, axis=0, keepdims=True)
    st_ref[1:2, :] += jnp.sum(y * y, axis=0, keepdims=True)


def _mlp1_body(x_ref, a_ref, c_ref, w_ref, b_ref, y_ref, st_ref):
    step = pl.program_id(0)

    @pl.when(step == 0)
    def _():
        st_ref[...] = jnp.zeros_like(st_ref)

    x = jax.nn.relu(x_ref[...] * a_ref[...] + c_ref[...])
    y = jnp.dot(x, w_ref[...], preferred_element_type=jnp.float32,
                precision=_PREC) + b_ref[...]
    y_ref[...] = y
    st_ref[0:1, :] += jnp.sum(y, axis=0, keepdims=True)
    st_ref[1:2, :] += jnp.sum(y * y, axis=0, keepdims=True)


def _mlp2_body(x_ref, a_ref, c_ref, w_ref, b_ref, ymax_ref, ymin_ref, st_ref):
    b = pl.program_id(0)
    s = pl.program_id(1)

    @pl.when((b == 0) & (s == 0))
    def _():
        st_ref[...] = jnp.zeros_like(st_ref)

    x = jax.nn.relu(x_ref[...] * a_ref[...] + c_ref[...])
    y = jnp.dot(x, w_ref[...], preferred_element_type=jnp.float32,
                precision=_PREC) + b_ref[...]

    @pl.when(s == 0)
    def _():
        ymax_ref[...] = y
        ymin_ref[...] = y

    @pl.when(s > 0)
    def _():
        ymax_ref[...] = jnp.maximum(ymax_ref[...], y)
        ymin_ref[...] = jnp.minimum(ymin_ref[...], y)

    st_ref[0:1, :] += jnp.sum(y, axis=0, keepdims=True)
    st_ref[1:2, :] += jnp.sum(y * y, axis=0, keepdims=True)


def _finish_body(ymax_ref, ymin_ref, a_ref, c_ref, out_ref):
    a = a_ref[...]
    c = c_ref[...]
    hi = jax.nn.relu(a * ymax_ref[...] + c)
    lo = jax.nn.relu(a * ymin_ref[...] + c)
    out_ref[...] = jnp.where(a > 0, hi, lo)


def _norm_coeffs(st, count, g, be):
    mean = st[0] / count
    var = st[1] / count - mean * mean
    a = g * lax.rsqrt(var + EPS)
    c = be - mean * a
    return a.reshape(1, -1), c.reshape(1, -1)


def kernel(xyz_proj, points_proj, xyz_sampled_proj, W0, b0, g0, be0,
           W1, b1, g1, be1, W2, b2, g2, be2):
    B, H, W, _ = xyz_proj.shape
    C = points_proj.shape[3]
    h, w = xyz_sampled_proj.shape[1], xyz_sampled_proj.shape[2]
    n = h * w
    n_hw = H * W
    zero_row = B * n_hw
    R = B * n * K_SAMPLE
    D1 = W1.shape[1]
    D2 = W2.shape[1]
    CP = 128

    planes = []
    for pr in (0, 1):
        for pc in (0, 1):
            q = xyz_proj[:, pr::2, pc::2, :]
            planes.append(jnp.moveaxis(q, -1, 1))
    xp = jnp.stack(planes, 1).reshape(B, 12, h, w)
    xp = jnp.pad(xp, ((0, 0), (0, 0), (2, 2), (2, 2)))

    idx = pl.pallas_call(
        functools.partial(_select_body, OH=h, OW=w, W_in=W, H_in=H,
                          zero_row=zero_row, n_hw=n_hw),
        grid=(B,),
        in_specs=[pl.BlockSpec((1, 12, h + 4, w + 4), lambda b: (b, 0, 0, 0))],
        out_specs=pl.BlockSpec((1, K_SAMPLE, h, w), lambda b: (b, 0, 0, 0)),
        out_shape=jax.ShapeDtypeStruct((B, K_SAMPLE, h, w), jnp.int32),
    )(xp)
    idx_flat = idx.reshape(R)

    table = jnp.concatenate(
        [points_proj.reshape(B * n_hw, C),
         xyz_proj.reshape(B * n_hw, 3),
         jnp.zeros((B * n_hw, CP - C - 3), jnp.float32)], axis=1)
    table = jnp.concatenate([table, jnp.zeros((8, CP), jnp.float32)], axis=0)
    G = _sc_gather(table, idx_flat, R, CP)

    W80 = jnp.concatenate(
        [W0[3:], W0[:3], jnp.zeros((CP - C - 3, W0.shape[1]), jnp.float32)], axis=0)
    nx = xyz_sampled_proj.reshape(B * n, 3)
    D0 = W0.shape[1]
    y0, st0 = pl.pallas_call(
        _mlp0_body,
        grid=(B, K_SAMPLE),
        in_specs=[
            pl.BlockSpec((n, CP), lambda b, s: (b * K_SAMPLE + s, 0)),
            pl.BlockSpec((n, 3), lambda b, s: (b, 0)),
            pl.BlockSpec((CP, D0), lambda b, s: (0, 0)),
            pl.BlockSpec((3, D0), lambda b, s: (0, 0)),
            pl.BlockSpec((1, D0), lambda b, s: (0, 0)),
        ],
        out_specs=[
            pl.BlockSpec((n, D0), lambda b, s: (b * K_SAMPLE + s, 0)),
            pl.BlockSpec((2, D0), lambda b, s: (0, 0)),
        ],
        out_shape=[
            jax.ShapeDtypeStruct((R, D0), jnp.float32),
            jax.ShapeDtypeStruct((2, D0), jnp.float32),
        ],
    )(G, nx, W80, W0[:3], b0.reshape(1, D0))
    a0, c0 = _norm_coeffs(st0, R, g0, be0)

    y1, st1 = pl.pallas_call(
        _mlp1_body,
        grid=(B * K_SAMPLE,),
        in_specs=[
            pl.BlockSpec((n, D0), lambda i: (i, 0)),
            pl.BlockSpec((1, D0), lambda i: (0, 0)),
            pl.BlockSpec((1, D0), lambda i: (0, 0)),
            pl.BlockSpec((D0, D1), lambda i: (0, 0)),
            pl.BlockSpec((1, D1), lambda i: (0, 0)),
        ],
        out_specs=[
            pl.BlockSpec((n, D1), lambda i: (i, 0)),
            pl.BlockSpec((2, D1), lambda i: (0, 0)),
        ],
        out_shape=[
            jax.ShapeDtypeStruct((R, D1), jnp.float32),
            jax.ShapeDtypeStruct((2, D1), jnp.float32),
        ],
    )(y0, a0, c0, W1, b1.reshape(1, D1))
    a1, c1 = _norm_coeffs(st1, R, g1, be1)

    ymax, ymin, st2 = pl.pallas_call(
        _mlp2_body,
        grid=(B, K_SAMPLE),
        in_specs=[
            pl.BlockSpec((n, D1), lambda b, s: (b * K_SAMPLE + s, 0)),
            pl.BlockSpec((1, D1), lambda b, s: (0, 0)),
            pl.BlockSpec((1, D1), lambda b, s: (0, 0)),
            pl.BlockSpec((D1, D2), lambda b, s: (0, 0)),
            pl.BlockSpec((1, D2), lambda b, s: (0, 0)),
        ],
        out_specs=[
            pl.BlockSpec((n, D2), lambda b, s: (b, 0)),
            pl.BlockSpec((n, D2), lambda b, s: (b, 0)),
            pl.BlockSpec((2, D2), lambda b, s: (0, 0)),
        ],
        out_shape=[
            jax.ShapeDtypeStruct((B * n, D2), jnp.float32),
            jax.ShapeDtypeStruct((B * n, D2), jnp.float32),
            jax.ShapeDtypeStruct((2, D2), jnp.float32),
        ],
    )(y1, a1, c1, W2, b2.reshape(1, D2))
    a2, c2 = _norm_coeffs(st2, R, g2, be2)

    out = pl.pallas_call(
        _finish_body,
        grid=(B,),
        in_specs=[
            pl.BlockSpec((n, D2), lambda b: (b, 0)),
            pl.BlockSpec((n, D2), lambda b: (b, 0)),
            pl.BlockSpec((1, D2), lambda b: (0, 0)),
            pl.BlockSpec((1, D2), lambda b: (0, 0)),
        ],
        out_specs=pl.BlockSpec((n, D2), lambda b: (b, 0)),
        out_shape=jax.ShapeDtypeStruct((B * n, D2), jnp.float32),
    )(ymax, ymin, a2, c2)

    pds = out.reshape(B, n, D2)
    return pds, pds.reshape(B, h, w, D2)
